# baseline (speedup 1.0000x reference)
import math

import jax
import jax.numpy as jnp
from jax.experimental import pallas as pl
from jax.experimental.pallas import tpu as pltpu

PAD = 1
SMOOTH = 0.1
ROWS = 2048
V = 32000
RB = 1024
NR = ROWS // RB
VB = 6400
NV = V // VB
LANES = 128
NLT = VB // LANES

_S = SMOOTH / (V - 2)
_C = (1.0 - SMOOTH) * math.log(1.0 - SMOOTH) + SMOOTH * math.log(_S)
_MT = (1.0 - SMOOTH) / _S


def _xent_block(lp_ref, t_ref, out_ref, acc_ref):
    i = pl.program_id(0)
    j = pl.program_id(1)
    tb = t_ref[pl.ds(i * RB, RB), :]
    blk = lp_ref[:, :]
    partial = None
    for k in range(NLT):
        cols = j * VB + k * LANES + jax.lax.broadcasted_iota(jnp.int32, (1, LANES), 1)
        m = jnp.where(cols == tb, _MT, 1.0)
        if k == 0:
            m = jnp.where(cols == PAD, 0.0, m)
        tmp = blk[:, k * LANES:(k + 1) * LANES] * m
        partial = tmp if k == 0 else partial + tmp

    @pl.when(j == 0)
    def _init():
        acc_ref[pl.ds(i * RB, RB), :] = partial

    @pl.when(j > 0)
    def _accum():
        acc_ref[pl.ds(i * RB, RB), :] = acc_ref[pl.ds(i * RB, RB), :] + partial

    @pl.when((i == NR - 1) & (j == NV - 1))
    def _finish():
        t = t_ref[:, :]
        nonpad = (t != PAD).astype(jnp.float32)
        rowtot = jnp.sum(acc_ref[:, :], axis=1, keepdims=True)
        n = jnp.sum(nonpad)
        out_ref[0, 0] = _C * n - _S * jnp.sum(nonpad * rowtot)


def kernel(log_probs, trg):
    lp = log_probs.reshape(ROWS, V)
    t2 = trg.reshape(ROWS, 1)
    out = pl.pallas_call(
        _xent_block,
        grid=(NR, NV),
        in_specs=[
            pl.BlockSpec((RB, VB), lambda i, j: (i, j)),
            pl.BlockSpec((ROWS, 1), lambda i, j: (0, 0)),
        ],
        out_specs=pl.BlockSpec((1, 1), lambda i, j: (0, 0), memory_space=pltpu.MemorySpace.SMEM),
        out_shape=jax.ShapeDtypeStruct((1, 1), jnp.float32),
        scratch_shapes=[pltpu.VMEM((ROWS, LANES), jnp.float32)],
        compiler_params=pltpu.CompilerParams(
            dimension_semantics=("arbitrary", "arbitrary"),
        ),
    )(lp, t2)
    return out[0, 0]


# speedup vs baseline: 1.0303x; 1.0303x over previous
import math

import jax
import jax.numpy as jnp
from jax.experimental import pallas as pl
from jax.experimental.pallas import tpu as pltpu

PAD = 1
SMOOTH = 0.1
ROWS = 2048
V = 32000
RB = 2048
NR = ROWS // RB
VB = 3200
NV = V // VB
LANES = 128
NLT = VB // LANES

_S = SMOOTH / (V - 2)
_C = (1.0 - SMOOTH) * math.log(1.0 - SMOOTH) + SMOOTH * math.log(_S)
_MT = (1.0 - SMOOTH) / _S


def _xent_block(lp_ref, t_ref, out_ref, acc_ref):
    i = pl.program_id(0)
    j = pl.program_id(1)
    tb = t_ref[pl.ds(i * RB, RB), :]
    blk = lp_ref[:, :]
    partial = None
    for k in range(NLT):
        cols = j * VB + k * LANES + jax.lax.broadcasted_iota(jnp.int32, (1, LANES), 1)
        m = jnp.where(cols == tb, _MT, 1.0)
        if k == 0:
            m = jnp.where(cols == PAD, 0.0, m)
        tmp = blk[:, k * LANES:(k + 1) * LANES] * m
        partial = tmp if k == 0 else partial + tmp

    @pl.when(j == 0)
    def _init():
        acc_ref[pl.ds(i * RB, RB), :] = partial

    @pl.when(j > 0)
    def _accum():
        acc_ref[pl.ds(i * RB, RB), :] = acc_ref[pl.ds(i * RB, RB), :] + partial

    @pl.when((i == NR - 1) & (j == NV - 1))
    def _finish():
        t = t_ref[:, :]
        nonpad = (t != PAD).astype(jnp.float32)
        rowtot = jnp.sum(acc_ref[:, :], axis=1, keepdims=True)
        n = jnp.sum(nonpad)
        out_ref[0, 0] = _C * n - _S * jnp.sum(nonpad * rowtot)


def kernel(log_probs, trg):
    lp = log_probs.reshape(ROWS, V)
    t2 = trg.reshape(ROWS, 1)
    out = pl.pallas_call(
        _xent_block,
        grid=(NR, NV),
        in_specs=[
            pl.BlockSpec((RB, VB), lambda i, j: (i, j)),
            pl.BlockSpec((ROWS, 1), lambda i, j: (0, 0)),
        ],
        out_specs=pl.BlockSpec((1, 1), lambda i, j: (0, 0), memory_space=pltpu.MemorySpace.SMEM),
        out_shape=jax.ShapeDtypeStruct((1, 1), jnp.float32),
        scratch_shapes=[pltpu.VMEM((ROWS, LANES), jnp.float32)],
        compiler_params=pltpu.CompilerParams(
            dimension_semantics=("arbitrary", "arbitrary"),
        ),
    )(lp, t2)
    return out[0, 0]


# speedup vs baseline: 1.0552x; 1.0241x over previous
import math

import jax
import jax.numpy as jnp
from jax.experimental import pallas as pl
from jax.experimental.pallas import tpu as pltpu

PAD = 1
SMOOTH = 0.1
ROWS = 2048
V = 32000
RB = 1024
NR = ROWS // RB
VB = 3200
NV = V // VB
LANES = 128
NLT = VB // LANES

_S = SMOOTH / (V - 2)
_C = (1.0 - SMOOTH) * math.log(1.0 - SMOOTH) + SMOOTH * math.log(_S)
_MT = (1.0 - SMOOTH) / _S


def _xent_block(lp_ref, t_ref, out_ref, acc_ref):
    i = pl.program_id(0)
    j = pl.program_id(1)
    tb = t_ref[pl.ds(i * RB, RB), :]
    blk = lp_ref[:, :]
    partial = None
    for k in range(NLT):
        cols = j * VB + k * LANES + jax.lax.broadcasted_iota(jnp.int32, (1, LANES), 1)
        m = jnp.where(cols == tb, _MT, 1.0)
        if k == 0:
            m = jnp.where(cols == PAD, 0.0, m)
        tmp = blk[:, k * LANES:(k + 1) * LANES] * m
        partial = tmp if k == 0 else partial + tmp

    @pl.when(j == 0)
    def _init():
        acc_ref[pl.ds(i * RB, RB), :] = partial

    @pl.when(j > 0)
    def _accum():
        acc_ref[pl.ds(i * RB, RB), :] = acc_ref[pl.ds(i * RB, RB), :] + partial

    @pl.when((i == NR - 1) & (j == NV - 1))
    def _finish():
        t = t_ref[:, :]
        nonpad = (t != PAD).astype(jnp.float32)
        rowtot = jnp.sum(acc_ref[:, :], axis=1, keepdims=True)
        n = jnp.sum(nonpad)
        out_ref[0, 0] = _C * n - _S * jnp.sum(nonpad * rowtot)


def kernel(log_probs, trg):
    lp = log_probs.reshape(ROWS, V)
    t2 = trg.reshape(ROWS, 1)
    out = pl.pallas_call(
        _xent_block,
        grid=(NR, NV),
        in_specs=[
            pl.BlockSpec((RB, VB), lambda i, j: (i, j)),
            pl.BlockSpec((ROWS, 1), lambda i, j: (0, 0)),
        ],
        out_specs=pl.BlockSpec((1, 1), lambda i, j: (0, 0), memory_space=pltpu.MemorySpace.SMEM),
        out_shape=jax.ShapeDtypeStruct((1, 1), jnp.float32),
        scratch_shapes=[pltpu.VMEM((ROWS, LANES), jnp.float32)],
        compiler_params=pltpu.CompilerParams(
            dimension_semantics=("arbitrary", "arbitrary"),
        ),
    )(lp, t2)
    return out[0, 0]
